# baseline (speedup 1.0000x reference)
import functools

import jax
import jax.numpy as jnp
from jax import lax
from jax.experimental import pallas as pl
from jax.experimental.pallas import tpu as pltpu
from jax.experimental.pallas import tpu_sc as plsc

NC = 2
NS = 16
NW = NC * NS


DEG_NBUF = 6


def _deg_body(nsc, per_sc, chunk, n_chunks, n_extra, col_hbm, ones_hbm,
              z_hbm, deg_out, ones_v, cidxs, xidx_v, deg_sh, isems, ssems,
              xssem):
    c = lax.axis_index("c")
    s = lax.axis_index("s")
    base = c * per_sc + (s * n_chunks + jnp.minimum(s, n_extra)) * chunk
    rpt = nsc // NS
    rounds = n_chunks // DEG_NBUF

    pltpu.sync_copy(ones_hbm, ones_v)
    pltpu.sync_copy(z_hbm.at[pl.ds(s * rpt, rpt)],
                    deg_sh.at[pl.ds(s * rpt, rpt)])
    plsc.subcore_barrier()

    def issue(j, b):
        off = base + (j * DEG_NBUF + b) * chunk
        pltpu.async_copy(col_hbm.at[pl.ds(off, chunk)], cidxs[b], isems[b])

    def fire(b):
        pltpu.make_async_copy(col_hbm.at[pl.ds(0, chunk)], cidxs[b],
                              isems[b]).wait()
        pltpu.async_copy(ones_v, deg_sh.at[cidxs[b]], ssems[b], add=True)

    def wait_scatter(b):
        pltpu.make_async_copy(col_hbm.at[pl.ds(0, chunk)], cidxs[b],
                              ssems[b]).wait()

    for b in range(DEG_NBUF):
        issue(0, b)

    def body(j, carry):
        for b in range(DEG_NBUF):
            fire(b)
        for b in range(DEG_NBUF):
            wait_scatter(b)
            issue(j + 1, b)
        return carry

    lax.fori_loop(0, rounds - 1, body, 0, unroll=False)
    for b in range(DEG_NBUF):
        fire(b)
    for b in range(DEG_NBUF):
        wait_scatter(b)
    for t in range(n_chunks - rounds * DEG_NBUF):
        issue(rounds, t)
        fire(t)
        wait_scatter(t)

    @pl.when(s < n_extra)
    def _extra():
        off = base + n_chunks * chunk
        pltpu.sync_copy(col_hbm.at[pl.ds(off, chunk)], xidx_v)
        pltpu.async_copy(ones_v, deg_sh.at[xidx_v], xssem, add=True)
        pltpu.make_async_copy(col_hbm.at[pl.ds(0, chunk)], xidx_v,
                              xssem).wait()

    plsc.subcore_barrier()

    pltpu.sync_copy(deg_sh.at[pl.ds(s * rpt, rpt)],
                    deg_out.at[pl.ds(c * nsc + s * rpt, rpt)])


NBUF = 3


def _agg_body(nsc, d, per_sc, chunk, n_chunks, n_extra, row_hbm, col_hbm,
              hp_hbm, z_hbm, s_out, ridx_v, cidxs, rowbufs, acc_sh, isems,
              gsems, ssems):
    c = lax.axis_index("c")
    s = lax.axis_index("s")
    base = c * per_sc + (s * n_chunks + jnp.minimum(s, n_extra)) * chunk
    my_chunks = n_chunks
    rows_per_tile = nsc // NS
    rounds = n_chunks // NBUF

    pltpu.sync_copy(z_hbm.at[pl.ds(s * rows_per_tile, rows_per_tile)],
                    acc_sh.at[pl.ds(s * rows_per_tile, rows_per_tile)])
    pltpu.sync_copy(row_hbm.at[pl.ds(base, n_chunks * chunk)], ridx_v)
    plsc.subcore_barrier()

    def issue(k, b):
        off = k * chunk
        pltpu.async_copy(col_hbm.at[pl.ds(base + off, chunk)], cidxs[b],
                         isems[b])
        pltpu.async_copy(hp_hbm.at[ridx_v.at[pl.ds(off, chunk)]], rowbufs[b],
                         gsems[b])

    def wait_gather(b):
        pltpu.make_async_copy(hp_hbm.at[pl.ds(0, chunk)], rowbufs[b],
                              gsems[b]).wait()

    def wait_idx(b):
        pltpu.make_async_copy(col_hbm.at[pl.ds(0, chunk)], cidxs[b],
                              isems[b]).wait()

    def wait_scatter(b):
        pltpu.make_async_copy(hp_hbm.at[pl.ds(0, chunk)], rowbufs[b],
                              ssems[b]).wait()

    def fire(b):
        wait_gather(b)
        wait_idx(b)
        pltpu.async_copy(rowbufs[b], acc_sh.at[cidxs[b]], ssems[b], add=True)

    for b in range(NBUF):
        issue(b, b)

    def body(j, carry):
        for b in range(NBUF):
            fire(b)
        for b in range(NBUF):
            wait_scatter(b)
            issue((j + 1) * NBUF + b, b)
        return carry

    lax.fori_loop(0, rounds - 1, body, 0, unroll=False)
    for b in range(NBUF):
        fire(b)
    for b in range(NBUF):
        wait_scatter(b)
    for t in range(my_chunks - rounds * NBUF):
        issue(rounds * NBUF + t, t)
        fire(t)
        wait_scatter(t)

    @pl.when(s < n_extra)
    def _extra():
        off = base + n_chunks * chunk
        pltpu.async_copy(col_hbm.at[pl.ds(off, chunk)], cidxs[0], isems[0])
        pltpu.sync_copy(row_hbm.at[pl.ds(off, chunk)],
                        ridx_v.at[pl.ds(0, chunk)])
        pltpu.async_copy(hp_hbm.at[ridx_v.at[pl.ds(0, chunk)]], rowbufs[0],
                         gsems[0])
        fire(0)
        wait_scatter(0)

    plsc.subcore_barrier()

    pltpu.sync_copy(acc_sh.at[pl.ds(s * rows_per_tile, rows_per_tile)],
                    s_out.at[c, pl.ds(s * rows_per_tile, rows_per_tile)])


def _tc_scale_body(x_ref, w_ref, degp_ref, hp_ref, dis_ref):
    deg = degp_ref[0, :, 0:1] + degp_ref[1, :, 0:1] + 1.0
    dis = lax.rsqrt(deg)
    h = jnp.dot(x_ref[...], w_ref[...], preferred_element_type=jnp.float32)
    hp_ref[...] = dis * h
    dis_ref[...] = dis


def _tc_combine_body(sp_ref, hp_ref, dis_ref, b_ref, o_ref):
    t = sp_ref[0] + sp_ref[1] + hp_ref[...]
    o_ref[...] = jnp.maximum(dis_ref[...] * t + b_ref[...], 0.0)


def kernel(x, edge_index, W, b):
    n, d_in = x.shape
    d_out = W.shape[1]
    e = edge_index.shape[1]
    assert e % NC == 0
    per_sc = e // NC
    chunk = 128
    assert per_sc % chunk == 0
    chunks_per_sc = per_sc // chunk
    n_chunks = chunks_per_sc // NS
    n_extra = chunks_per_sc % NS
    np_ = ((n + 16 * NS - 1) // (16 * NS)) * (16 * NS)

    row = edge_index[0]
    col = edge_index[1]
    xp = jnp.pad(x, ((0, np_ - n), (0, 0)))
    ones1 = jnp.ones((chunk,), jnp.float32)
    z1 = jnp.zeros((np_,), jnp.float32)
    zfull = jnp.zeros((np_, d_out), jnp.float32)

    mesh = plsc.VectorSubcoreMesh(core_axis_name="c", subcore_axis_name="s")

    degp = pl.kernel(
        functools.partial(_deg_body, np_, per_sc, chunk, n_chunks, n_extra),
        out_type=jax.ShapeDtypeStruct((NC * np_,), jnp.float32),
        mesh=mesh,
        scratch_types=[
            pltpu.VMEM((chunk,), jnp.float32),
            [pltpu.VMEM((chunk,), jnp.int32) for _ in range(DEG_NBUF)],
            pltpu.VMEM((chunk,), jnp.int32),
            pltpu.VMEM_SHARED((np_,), jnp.float32),
            [pltpu.SemaphoreType.DMA for _ in range(DEG_NBUF)],
            [pltpu.SemaphoreType.DMA for _ in range(DEG_NBUF)],
            pltpu.SemaphoreType.DMA,
        ],
    )(col, ones1, z1)

    grid_s = 10
    sblk = np_ // grid_s
    hp, dis = pl.pallas_call(
        _tc_scale_body,
        grid=(grid_s,),
        in_specs=[
            pl.BlockSpec((sblk, d_in), lambda i: (i, 0)),
            pl.BlockSpec((d_in, d_out), lambda i: (0, 0)),
            pl.BlockSpec((NC, sblk, 1), lambda i: (0, i, 0)),
        ],
        out_specs=[
            pl.BlockSpec((sblk, d_out), lambda i: (i, 0)),
            pl.BlockSpec((sblk, 1), lambda i: (i, 0)),
        ],
        out_shape=[
            jax.ShapeDtypeStruct((np_, d_out), jnp.float32),
            jax.ShapeDtypeStruct((np_, 1), jnp.float32),
        ],
    )(xp, W, degp.reshape(NC, np_, 1))

    cchunk = 80
    assert per_sc % (cchunk * NS) == 0
    cn_chunks = per_sc // cchunk // NS
    sp = pl.kernel(
        functools.partial(_agg_body, np_, d_out, per_sc, cchunk, cn_chunks,
                          0),
        out_type=jax.ShapeDtypeStruct((NC, np_, d_out), jnp.float32),
        mesh=mesh,
        scratch_types=[
            pltpu.VMEM((cn_chunks * cchunk,), jnp.int32),
            [pltpu.VMEM((cchunk,), jnp.int32) for _ in range(NBUF)],
            [pltpu.VMEM((cchunk, d_out), jnp.float32) for _ in range(NBUF)],
            pltpu.VMEM_SHARED((np_, d_out), jnp.float32),
            [pltpu.SemaphoreType.DMA for _ in range(NBUF)],
            [pltpu.SemaphoreType.DMA for _ in range(NBUF)],
            [pltpu.SemaphoreType.DMA for _ in range(NBUF)],
        ],
    )(row, col, hp, zfull)

    grid_d = 10
    blk = n // grid_d
    out = pl.pallas_call(
        _tc_combine_body,
        grid=(grid_d,),
        in_specs=[
            pl.BlockSpec((NC, blk, d_out), lambda i: (0, i, 0)),
            pl.BlockSpec((blk, d_out), lambda i: (i, 0)),
            pl.BlockSpec((blk, 1), lambda i: (i, 0)),
            pl.BlockSpec((1, d_out), lambda i: (0, 0)),
        ],
        out_specs=pl.BlockSpec((blk, d_out), lambda i: (i, 0)),
        out_shape=jax.ShapeDtypeStruct((n, d_out), jnp.float32),
    )(sp, hp, dis, b.reshape(1, d_out))
    return out


# speedup vs baseline: 1.0096x; 1.0096x over previous
import functools

import jax
import jax.numpy as jnp
from jax import lax
from jax.experimental import pallas as pl
from jax.experimental.pallas import tpu as pltpu
from jax.experimental.pallas import tpu_sc as plsc

NC = 2
NS = 16
NW = NC * NS


DEG_NBUF = 6


def _deg_body(nsc, per_sc, chunk, n_chunks, n_extra, col_hbm, ones_hbm,
              z_hbm, deg_out, ones_v, cidxs, xidx_v, deg_sh, isems, ssems,
              xssem):
    c = lax.axis_index("c")
    s = lax.axis_index("s")
    base = c * per_sc + (s * n_chunks + jnp.minimum(s, n_extra)) * chunk
    rpt = nsc // NS
    rounds = n_chunks // DEG_NBUF

    pltpu.sync_copy(ones_hbm, ones_v)
    pltpu.sync_copy(z_hbm.at[pl.ds(s * rpt, rpt)],
                    deg_sh.at[pl.ds(s * rpt, rpt)])
    plsc.subcore_barrier()

    def issue(j, b):
        off = base + (j * DEG_NBUF + b) * chunk
        pltpu.async_copy(col_hbm.at[pl.ds(off, chunk)], cidxs[b], isems[b])

    def fire(b):
        pltpu.make_async_copy(col_hbm.at[pl.ds(0, chunk)], cidxs[b],
                              isems[b]).wait()
        pltpu.async_copy(ones_v, deg_sh.at[cidxs[b]], ssems[b], add=True)

    def wait_scatter(b):
        pltpu.make_async_copy(col_hbm.at[pl.ds(0, chunk)], cidxs[b],
                              ssems[b]).wait()

    for b in range(DEG_NBUF):
        issue(0, b)

    def body(j, carry):
        for b in range(DEG_NBUF):
            fire(b)
        for b in range(DEG_NBUF):
            wait_scatter(b)
            issue(j + 1, b)
        return carry

    lax.fori_loop(0, rounds - 1, body, 0, unroll=False)
    for b in range(DEG_NBUF):
        fire(b)
    for b in range(DEG_NBUF):
        wait_scatter(b)
    for t in range(n_chunks - rounds * DEG_NBUF):
        issue(rounds, t)
        fire(t)
        wait_scatter(t)

    @pl.when(s < n_extra)
    def _extra():
        off = base + n_chunks * chunk
        pltpu.sync_copy(col_hbm.at[pl.ds(off, chunk)], xidx_v)
        pltpu.async_copy(ones_v, deg_sh.at[xidx_v], xssem, add=True)
        pltpu.make_async_copy(col_hbm.at[pl.ds(0, chunk)], xidx_v,
                              xssem).wait()

    plsc.subcore_barrier()

    pltpu.sync_copy(deg_sh.at[pl.ds(s * rpt, rpt)],
                    deg_out.at[pl.ds(c * nsc + s * rpt, rpt)])


NBUF = 3


def _agg_body(nsc, d, per_sc, chunk, n_chunks, n_extra, row_hbm, col_hbm,
              hp_hbm, z_hbm, s_out, ridx_v, cidxs, rowbufs, acc_sh, isems,
              gsems, ssems):
    c = lax.axis_index("c")
    s = lax.axis_index("s")
    base = c * per_sc + (s * n_chunks + jnp.minimum(s, n_extra)) * chunk
    my_chunks = n_chunks
    rows_per_tile = nsc // NS
    rounds = n_chunks // NBUF

    pltpu.sync_copy(z_hbm.at[pl.ds(s * rows_per_tile, rows_per_tile)],
                    acc_sh.at[pl.ds(s * rows_per_tile, rows_per_tile)])
    pltpu.sync_copy(row_hbm.at[pl.ds(base, n_chunks * chunk)], ridx_v)
    plsc.subcore_barrier()

    def issue(k, b):
        off = k * chunk
        pltpu.async_copy(col_hbm.at[pl.ds(base + off, chunk)], cidxs[b],
                         isems[b])
        pltpu.async_copy(hp_hbm.at[ridx_v.at[pl.ds(off, chunk)]], rowbufs[b],
                         gsems[b])

    def wait_gather(b):
        pltpu.make_async_copy(hp_hbm.at[pl.ds(0, chunk)], rowbufs[b],
                              gsems[b]).wait()

    def wait_idx(b):
        pltpu.make_async_copy(col_hbm.at[pl.ds(0, chunk)], cidxs[b],
                              isems[b]).wait()

    def wait_scatter(b):
        pltpu.make_async_copy(hp_hbm.at[pl.ds(0, chunk)], rowbufs[b],
                              ssems[b]).wait()

    def fire(b):
        wait_gather(b)
        wait_idx(b)
        pltpu.async_copy(rowbufs[b], acc_sh.at[cidxs[b]], ssems[b], add=True)

    for b in range(NBUF):
        issue(b, b)

    def body(j, carry):
        for b in range(NBUF):
            fire(b)
        for b in range(NBUF):
            wait_scatter(b)
            issue((j + 1) * NBUF + b, b)
        return carry

    lax.fori_loop(0, rounds - 1, body, 0, unroll=False)
    for b in range(NBUF):
        fire(b)
    for b in range(NBUF):
        wait_scatter(b)
    for t in range(my_chunks - rounds * NBUF):
        issue(rounds * NBUF + t, t)
        fire(t)
        wait_scatter(t)

    @pl.when(s < n_extra)
    def _extra():
        off = base + n_chunks * chunk
        pltpu.async_copy(col_hbm.at[pl.ds(off, chunk)], cidxs[0], isems[0])
        pltpu.sync_copy(row_hbm.at[pl.ds(off, chunk)],
                        ridx_v.at[pl.ds(0, chunk)])
        pltpu.async_copy(hp_hbm.at[ridx_v.at[pl.ds(0, chunk)]], rowbufs[0],
                         gsems[0])
        fire(0)
        wait_scatter(0)

    plsc.subcore_barrier()

    pltpu.sync_copy(acc_sh.at[pl.ds(s * rows_per_tile, rows_per_tile)],
                    s_out.at[c, pl.ds(s * rows_per_tile, rows_per_tile)])


def _tc_scale_body(x_ref, w_ref, degp_ref, hp_ref, dis_ref):
    deg = degp_ref[0, :, 0:1] + degp_ref[1, :, 0:1] + 1.0
    dis = lax.rsqrt(deg)
    h = jnp.dot(x_ref[...], w_ref[...], preferred_element_type=jnp.float32)
    hp_ref[...] = dis * h
    dis_ref[...] = dis


def _tc_combine_body(sp_ref, hp_ref, dis_ref, b_ref, o_ref):
    t = sp_ref[0] + sp_ref[1] + hp_ref[...]
    o_ref[...] = jnp.maximum(dis_ref[...] * t + b_ref[...], 0.0)


def kernel(x, edge_index, W, b):
    n, d_in = x.shape
    d_out = W.shape[1]
    e = edge_index.shape[1]
    assert e % NC == 0
    per_sc = e // NC
    chunk = 128
    assert per_sc % chunk == 0
    chunks_per_sc = per_sc // chunk
    n_chunks = chunks_per_sc // NS
    n_extra = chunks_per_sc % NS
    np_ = ((n + 16 * NS - 1) // (16 * NS)) * (16 * NS)

    row = edge_index[0]
    col = edge_index[1]
    xp = jnp.pad(x, ((0, np_ - n), (0, 0)))
    ones1 = jnp.ones((chunk,), jnp.float32)
    z1 = jnp.zeros((np_,), jnp.float32)
    zfull = jnp.zeros((np_, d_out), jnp.float32)

    mesh = plsc.VectorSubcoreMesh(core_axis_name="c", subcore_axis_name="s")

    degp = pl.kernel(
        functools.partial(_deg_body, np_, per_sc, chunk, n_chunks, n_extra),
        out_type=jax.ShapeDtypeStruct((NC * np_,), jnp.float32),
        mesh=mesh,
        scratch_types=[
            pltpu.VMEM((chunk,), jnp.float32),
            [pltpu.VMEM((chunk,), jnp.int32) for _ in range(DEG_NBUF)],
            pltpu.VMEM((chunk,), jnp.int32),
            pltpu.VMEM_SHARED((np_,), jnp.float32),
            [pltpu.SemaphoreType.DMA for _ in range(DEG_NBUF)],
            [pltpu.SemaphoreType.DMA for _ in range(DEG_NBUF)],
            pltpu.SemaphoreType.DMA,
        ],
    )(col, ones1, z1)

    hp, dis = pl.pallas_call(
        _tc_scale_body,
        out_shape=[
            jax.ShapeDtypeStruct((np_, d_out), jnp.float32),
            jax.ShapeDtypeStruct((np_, 1), jnp.float32),
        ],
    )(xp, W, degp.reshape(NC, np_, 1))

    cchunk = 80
    assert per_sc % (cchunk * NS) == 0
    cn_chunks = per_sc // cchunk // NS
    sp = pl.kernel(
        functools.partial(_agg_body, np_, d_out, per_sc, cchunk, cn_chunks,
                          0),
        out_type=jax.ShapeDtypeStruct((NC, np_, d_out), jnp.float32),
        mesh=mesh,
        scratch_types=[
            pltpu.VMEM((cn_chunks * cchunk,), jnp.int32),
            [pltpu.VMEM((cchunk,), jnp.int32) for _ in range(NBUF)],
            [pltpu.VMEM((cchunk, d_out), jnp.float32) for _ in range(NBUF)],
            pltpu.VMEM_SHARED((np_, d_out), jnp.float32),
            [pltpu.SemaphoreType.DMA for _ in range(NBUF)],
            [pltpu.SemaphoreType.DMA for _ in range(NBUF)],
            [pltpu.SemaphoreType.DMA for _ in range(NBUF)],
        ],
    )(row, col, hp, zfull)

    grid_d = 10
    blk = n // grid_d
    out = pl.pallas_call(
        _tc_combine_body,
        grid=(grid_d,),
        in_specs=[
            pl.BlockSpec((NC, blk, d_out), lambda i: (0, i, 0)),
            pl.BlockSpec((blk, d_out), lambda i: (i, 0)),
            pl.BlockSpec((blk, 1), lambda i: (i, 0)),
            pl.BlockSpec((1, d_out), lambda i: (0, 0)),
        ],
        out_specs=pl.BlockSpec((blk, d_out), lambda i: (i, 0)),
        out_shape=jax.ShapeDtypeStruct((n, d_out), jnp.float32),
    )(sp, hp, dis, b.reshape(1, d_out))
    return out
